# baseline (speedup 1.0000x reference)
import functools

import jax
import jax.numpy as jnp
from jax.experimental import pallas as pl


M_TILE = 512
N_ROWS_K = 16384
D_K = 2048


def _mlp_kernel(x_ref, w1_ref, b1_ref, w2_ref, b2_ref, out_ref):
    x = x_ref[...].astype(jnp.bfloat16)
    h = jnp.dot(x, w1_ref[...], preferred_element_type=jnp.float32)
    h = jnp.maximum(h + b1_ref[...], 0.0)
    logits = jnp.sum(h * w2_ref[...], axis=1) + b2_ref[0, 0]
    out_ref[0, 0, :] = jax.nn.sigmoid(logits)


@functools.partial(jax.jit, static_argnames=())
def kernel(img_features, W1, b1, W2, b2):
    n, d = img_features.shape
    num_tiles = n // M_TILE
    w1b = W1.astype(jnp.bfloat16)
    b1r = b1.reshape(1, d)
    w2r = W2.reshape(1, d)
    b2r = b2.reshape(1, 1)
    out = pl.pallas_call(
        _mlp_kernel,
        grid=(num_tiles,),
        in_specs=[
            pl.BlockSpec((M_TILE, d), lambda i: (i, 0)),
            pl.BlockSpec((d, d), lambda i: (0, 0)),
            pl.BlockSpec((1, d), lambda i: (0, 0)),
            pl.BlockSpec((1, d), lambda i: (0, 0)),
            pl.BlockSpec((1, 1), lambda i: (0, 0)),
        ],
        out_specs=pl.BlockSpec((1, 1, M_TILE), lambda i: (i, 0, 0)),
        out_shape=jax.ShapeDtypeStruct((num_tiles, 1, M_TILE), jnp.float32),
    )(img_features, w1b, b1r, w2r, b2r)
    return out.reshape(n, 1)
